# baseline (device time: 23234 ns/iter reference)
import jax
import jax.numpy as jnp
from jax import lax
from jax.experimental import pallas as pl
from jax.experimental.pallas import tpu as pltpu

N_DEV = 32


def kernel(x, w_mat):
    m_per, k = x.shape
    n = w_mat.shape[1]
    n_per = n // N_DEV
    out_rows = N_DEV * m_per

    def body(x_ref, w_ref, out_ref, y_tiles, send_sems, recv_sems):
        my = lax.axis_index("i")

        barrier_sem = pltpu.get_barrier_semaphore()
        dsts = []
        for d in range(1, N_DEV):
            t = my + d
            dst = lax.select(t >= N_DEV, t - N_DEV, t)
            dsts.append(dst)
            pl.semaphore_signal(
                barrier_sem, inc=1,
                device_id=(dst,),
                device_id_type=pl.DeviceIdType.MESH,
            )

        y = jnp.dot(x_ref[...], w_ref[...], preferred_element_type=jnp.float32)
        y = (y * jax.nn.sigmoid(y)).astype(jnp.bfloat16)

        for p in range(N_DEV):
            y_tiles[p] = y[:, p * n_per:(p + 1) * n_per]

        my_rows = pl.ds(my * m_per, m_per)

        pl.semaphore_wait(barrier_sem, N_DEV - 1)

        rdmas = []
        for d in range(1, N_DEV):
            dst = dsts[d - 1]
            rdma = pltpu.make_async_remote_copy(
                src_ref=y_tiles.at[dst],
                dst_ref=out_ref.at[my_rows, :],
                send_sem=send_sems.at[d],
                recv_sem=recv_sems.at[d],
                device_id=(dst,),
                device_id_type=pl.DeviceIdType.MESH,
            )
            rdma.start()
            rdmas.append(rdma)

        out_ref[my_rows, :] = y_tiles[my]

        for rdma in rdmas:
            rdma.wait_send()
        for rdma in rdmas:
            rdma.wait_recv()

    return pl.pallas_call(
        body,
        out_shape=jax.ShapeDtypeStruct((out_rows, n_per), jnp.bfloat16),
        in_specs=[
            pl.BlockSpec(memory_space=pltpu.VMEM),
            pl.BlockSpec(memory_space=pltpu.VMEM),
        ],
        out_specs=pl.BlockSpec(memory_space=pltpu.VMEM),
        scratch_shapes=[
            pltpu.VMEM((N_DEV, m_per, n_per), jnp.bfloat16),
            pltpu.SemaphoreType.DMA((N_DEV,)),
            pltpu.SemaphoreType.DMA((N_DEV,)),
        ],
        compiler_params=pltpu.CompilerParams(collective_id=0),
    )(x, w_mat)
